# baseline (speedup 1.0000x reference)
import functools

import jax
import jax.numpy as jnp
from jax.experimental import pallas as pl
from jax.experimental.pallas import tpu as pltpu

B, S, D = 1, 2048, 1024
E, K, DFF = 8, 2, 1024

TS = 256


def _moe_dense_kernel(scores_ref, x_ref, w1_ref, w2_ref, w3_ref, out_ref):
    e = pl.program_id(1)

    scores = scores_ref[...]
    iota_e = jax.lax.broadcasted_iota(jnp.int32, (TS, E), 1)
    m1 = jnp.max(scores, axis=1, keepdims=True)
    i1 = jnp.min(jnp.where(scores == m1, iota_e, E), axis=1, keepdims=True)
    masked = jnp.where(iota_e == i1, -jnp.inf, scores)
    m2 = jnp.max(masked, axis=1, keepdims=True)
    i2 = jnp.min(jnp.where(masked == m2, iota_e, E), axis=1, keepdims=True)
    e2 = jnp.exp(m2 - m1)
    denom = 1.0 + e2
    p1 = 1.0 / denom
    p2 = e2 / denom
    g = jnp.where(i1 == e, p1, jnp.where(i2 == e, p2, 0.0))

    xb = x_ref[...].astype(jnp.bfloat16)
    a = jax.lax.dot_general(xb, w1_ref[0], (((1,), (1,)), ((), ())),
                            preferred_element_type=jnp.float32)
    b = jax.lax.dot_general(xb, w2_ref[0], (((1,), (1,)), ((), ())),
                            preferred_element_type=jnp.float32)
    h = (jax.nn.silu(a) * b).astype(jnp.bfloat16)
    contrib = jax.lax.dot_general(h, w3_ref[0], (((1,), (1,)), ((), ())),
                                  preferred_element_type=jnp.float32)
    contrib = g * contrib

    @pl.when(e == 0)
    def _():
        out_ref[...] = contrib

    @pl.when(e > 0)
    def _():
        out_ref[...] += contrib


@jax.jit
def kernel(x, Wg, W1, W2, W3):
    xs = x.reshape(S, D)
    scores = jnp.einsum('bsd,ed->bse', x, Wg).reshape(S, E)
    w1 = W1.astype(jnp.bfloat16)
    w2 = W2.astype(jnp.bfloat16)
    w3 = W3.astype(jnp.bfloat16)

    y = pl.pallas_call(
        _moe_dense_kernel,
        grid=(S // TS, E),
        in_specs=[
            pl.BlockSpec((TS, E), lambda i, e: (i, 0)),
            pl.BlockSpec((TS, D), lambda i, e: (i, 0)),
            pl.BlockSpec((1, DFF, D), lambda i, e: (e, 0, 0)),
            pl.BlockSpec((1, DFF, D), lambda i, e: (e, 0, 0)),
            pl.BlockSpec((1, D, DFF), lambda i, e: (e, 0, 0)),
        ],
        out_specs=pl.BlockSpec((TS, D), lambda i, e: (i, 0)),
        out_shape=jax.ShapeDtypeStruct((S, D), jnp.float32),
        compiler_params=pltpu.CompilerParams(
            dimension_semantics=("parallel", "arbitrary"),
        ),
    )(scores, xs, w1, w2, w3)
    return y.reshape(B, S, D)
